# baseline (speedup 1.0000x reference)
import functools

import jax
import jax.numpy as jnp
from jax import lax
from jax.experimental import pallas as pl
from jax.experimental.pallas import tpu as pltpu
from jax.experimental.pallas import tpu_sc as plsc


def _proj_body(x_ref, wy_ref, wr_ref, y_ref, xr_ref):
    x = x_ref[...]
    y_ref[...] = x @ wy_ref[...]
    xr_ref[...] = x @ wr_ref[...]


def _proj(x, w_cat, root, rows_blk):
    n, k = x.shape
    f = w_cat.shape[1]
    h = root.shape[1]
    grid = (n // rows_blk,)
    return pl.pallas_call(
        _proj_body,
        grid=grid,
        in_specs=[
            pl.BlockSpec((rows_blk, k), lambda i: (i, 0)),
            pl.BlockSpec((k, f), lambda i: (0, 0)),
            pl.BlockSpec((k, h), lambda i: (0, 0)),
        ],
        out_specs=[
            pl.BlockSpec((rows_blk, f), lambda i: (i, 0)),
            pl.BlockSpec((rows_blk, h), lambda i: (i, 0)),
        ],
        out_shape=[
            jax.ShapeDtypeStruct((n, f), jnp.float32),
            jax.ShapeDtypeStruct((n, h), jnp.float32),
        ],
    )(x, w_cat, root)


def _combine_body(do_relu, out_pad, r, h, xr_ref, acc_ref, cnt_ref, b_ref,
                  o_ref):
    rows = xr_ref.shape[0]
    inv = 1.0 / jnp.maximum(cnt_ref[...], 1.0)
    nslot = acc_ref.shape[1] // h
    acc = acc_ref[...].reshape(rows, nslot, h)
    agg = jnp.sum(acc[:, :r, :] * inv[:, :, None], axis=1)
    out = xr_ref[...] + b_ref[...] + agg
    if do_relu:
        out = jnp.maximum(out, 0.0)
    if out_pad:
        out = jnp.concatenate([out, jnp.zeros((rows, out_pad), jnp.float32)],
                              axis=1)
    o_ref[...] = out


def _combine(xroot, acc, cnt, b, do_relu, rows_blk, out_pad=0):
    n, h = xroot.shape
    r = cnt.shape[1]
    f = acc.shape[1]
    grid = (n // rows_blk,)
    return pl.pallas_call(
        functools.partial(_combine_body, do_relu, out_pad, r, h),
        grid=grid,
        in_specs=[
            pl.BlockSpec((rows_blk, h), lambda i: (i, 0)),
            pl.BlockSpec((rows_blk, f), lambda i: (i, 0)),
            pl.BlockSpec((rows_blk, r), lambda i: (i, 0)),
            pl.BlockSpec((1, h), lambda i: (0, 0)),
        ],
        out_specs=pl.BlockSpec((rows_blk, h + out_pad), lambda i: (i, 0)),
        out_shape=jax.ShapeDtypeStruct((n, h + out_pad), jnp.float32),
    )(xroot, acc, cnt, b.reshape(1, h))


def _head_body(nb, h, rows_ref, a1_ref, a2_ref, b1_ref, w2_ref, b2_ref,
               o_ref):
    big = rows_ref[...]
    bill = big[0:nb, 0:h]
    a1 = a1_ref[...]
    a2 = a2_ref[...]
    b1 = b1_ref[...]
    w2 = w2_ref[...]
    b2 = b2_ref[0, 0]

    def mlp(u):
        h1 = jnp.maximum(bill @ a1 + u @ a2 + b1, 0.0)
        return (h1 * w2).sum(axis=1, keepdims=True) + b2

    p = mlp(big[nb:2 * nb, 0:h])
    q = mlp(big[2 * nb:3 * nb, 0:h])
    t = (jnp.maximum(p, 0.0) - p + jnp.log1p(jnp.exp(-jnp.abs(p)))
         + jnp.maximum(q, 0.0) + jnp.log1p(jnp.exp(-jnp.abs(q))))
    o_ref[0, 0] = jnp.sum(t) / (2.0 * nb)


def _head(rows, fc1_w, fc1_b, fc2_w, fc2_b):
    nb = rows.shape[0] // 3
    h = fc1_w.shape[0]
    a = fc1_w.T
    a1, a2 = a[:h], a[h:]
    return pl.pallas_call(
        functools.partial(_head_body, nb, h),
        out_specs=pl.BlockSpec(memory_space=pltpu.SMEM),
        out_shape=jax.ShapeDtypeStruct((1, 1), jnp.float32),
    )(rows, a1, a2, fc1_b.reshape(1, h),
      fc2_w.reshape(1, h), fc2_b.reshape(1, 1))


_NB = 4096
_GPW = 3 * _NB // 32


def _gather_body(nodes_hbm, ids_hbm, out_hbm, idx_v, rows_v):
    wid = lax.axis_index("s") * 2 + lax.axis_index("c")
    base = wid * _GPW
    for k in range(_GPW // 128):
        pltpu.sync_copy(ids_hbm.at[pl.ds(base + k * 128, 128)], idx_v.at[k])
    for k in range(_GPW // 128):
        pltpu.sync_copy(nodes_hbm.at[idx_v.at[k]],
                        rows_v.at[pl.ds(k * 128, 128)])
    pltpu.sync_copy(rows_v, out_hbm.at[pl.ds(base, _GPW)])


def _id_gather(nodes128, ids):
    mesh = plsc.VectorSubcoreMesh(core_axis_name="c", subcore_axis_name="s")
    gath = pl.kernel(
        _gather_body,
        mesh=mesh,
        out_type=jax.ShapeDtypeStruct((3 * _NB, 128), jnp.float32),
        scratch_types=[
            pltpu.VMEM((_GPW // 128, 128), jnp.int32),
            pltpu.VMEM((_GPW, 128), jnp.float32),
        ],
    )
    return gath(nodes128, ids)


_N = 10000
_E = 320000
_R = 5
_H = 64
_NC = 2
_NS = 16
_NRG = 2560
_NPASS = 2
_NRANGE = _NC * _NPASS
_ACC_REAL = _NRG * 3
_TRASH = _ACC_REAL
_ACC_ROWS = 7696
_NHALF = _NRG * _NPASS
_CNT_REAL = _NHALF * _R
_TRASH_C = _CNT_REAL
_CNT_LEN = 25856
_CHUNK = 512
_CPT = 40
_EPT = _CHUNK * _CPT
_EPAD = _EPT * _NS
_NCH = _EPAD // _CHUNK
_KB = _CHUNK // 128


def _agg_body(y_hbm, e3_hbm, acc_out,
              e3_v, gidx_v, lidx_v, rows4_v, acc_sh, sem_g, sem_s):
    c = lax.axis_index("c")
    s = lax.axis_index("s")
    zf = jnp.zeros((16,), jnp.float32)
    base_c = s * _CPT

    for p in range(_NPASS):
        rg = c * _NPASS + p
        rg_lo = rg * _NRG

        def _zr(i, _):
            for k in range(8):
                rows4_v[i, pl.ds(k * 16, 16)] = zf
            return 0
        lax.fori_loop(0, 128, _zr, 0)
        for o in (0, 120, 240, 360):
            pltpu.sync_copy(rows4_v.at[pl.ds(0, 120)],
                            acc_sh.at[pl.ds(s * 480 + o, 120)])

        @pl.when(s == 0)
        def _():
            pltpu.sync_copy(rows4_v.at[pl.ds(0, 16)],
                            acc_sh.at[pl.ds(_TRASH, 16)])
        plsc.subcore_barrier()

        def _chunk(ch, _, p=p, rg_lo=rg_lo):
            pltpu.sync_copy(e3_hbm.at[pl.ds((base_c + ch) * 3 * _CHUNK,
                                            3 * _CHUNK)], e3_v)
            for j in range(_CHUNK // 16):
                s16 = e3_v[pl.ds(j * 16, 16)]
                d16 = e3_v[pl.ds(_CHUNK + j * 16, 16)]
                t16 = e3_v[pl.ds(2 * _CHUNK + j * 16, 16)]
                gi = s16 * _R + t16
                li = (d16 - rg_lo) * 3 + jnp.right_shift(t16, 1)
                m = (d16 >= rg_lo) & (d16 < rg_lo + _NRG)
                jsl = pl.ds((j % 8) * 16, 16)
                gidx_v[j // 8, jsl] = gi
                lidx_v[j // 8, jsl] = jnp.where(m, li, _TRASH + (j % 16))
            @pl.when(ch > 0)
            def _():
                pltpu.make_async_copy(y_hbm.at[pl.ds(0, _CHUNK)],
                                      rows4_v, sem_s).wait()
            hs = [pltpu.async_copy(y_hbm.at[gidx_v.at[k]],
                                   rows4_v.at[pl.ds(k * 128, 128)], sem_g)
                  for k in range(_CHUNK // 128)]
            for k, h in enumerate(hs):
                h.wait()
                pltpu.async_copy(rows4_v.at[pl.ds(k * 128, 128)],
                                 acc_sh.at[lidx_v.at[k]], sem_s, add=True)
            return 0
        lax.fori_loop(0, _CPT, _chunk, 0)
        pltpu.make_async_copy(y_hbm.at[pl.ds(0, _CHUNK)],
                              rows4_v, sem_s).wait()
        plsc.subcore_barrier()

        for o, w in ((0, 256), (256, 224)):
            pltpu.sync_copy(acc_sh.at[pl.ds(s * 480 + o, w)],
                            rows4_v.at[pl.ds(0, w)])
            pltpu.sync_copy(rows4_v.at[pl.ds(0, w)],
                            acc_out.at[rg, pl.ds(s * 480 + o, w)])
        plsc.subcore_barrier()


def _cnt_body(e3_hbm, cnt0_out, cnt1_out,
              e3_v, cidx_v, ones_v, zc_v, cnt_sh):
    c = lax.axis_index("c")
    s = lax.axis_index("s")
    zf = jnp.zeros((16,), jnp.float32)
    for k in range(8):
        ones_v[pl.ds(k * 16, 16)] = jnp.full((16,), 1.0, jnp.float32)

    def _zc(i, _):
        zc_v[pl.ds(i * 16, 16)] = zf
        return 0
    lax.fori_loop(0, 101, _zc, 0)
    pltpu.sync_copy(zc_v, cnt_sh.at[pl.ds(s * 1616, 1616)])
    plsc.subcore_barrier()

    chalf = c * _NHALF
    base_c = s * _CPT

    def _chunk(ch, _):
        pltpu.sync_copy(e3_hbm.at[pl.ds((base_c + ch) * 3 * _CHUNK,
                                        3 * _CHUNK)], e3_v)
        for j in range(_CHUNK // 16):
            d16 = e3_v[pl.ds(_CHUNK + j * 16, 16)]
            t16 = e3_v[pl.ds(2 * _CHUNK + j * 16, 16)]
            dc = d16 - chalf
            mc = (d16 >= chalf) & (d16 < chalf + _NHALF)
            ci = jnp.where(mc, dc * _R + t16, _TRASH_C)
            cidx_v[j // 8, pl.ds((j % 8) * 16, 16)] = ci
        for k in range(_CHUNK // 128):
            pltpu.sync_copy(ones_v, cnt_sh.at[cidx_v.at[k]], add=True)
        return 0
    lax.fori_loop(0, _CPT, _chunk, 0)
    plsc.subcore_barrier()

    def _cnt_export(cnt_out):
        pltpu.sync_copy(cnt_sh.at[pl.ds(s * 1600, 1600)],
                        zc_v.at[pl.ds(0, 1600)])
        pltpu.sync_copy(zc_v.at[pl.ds(0, 1600)],
                        cnt_out.at[pl.ds(s * 1600, 1600)])

    @pl.when(c == 0)
    def _():
        _cnt_export(cnt0_out)

    @pl.when(c == 1)
    def _():
        _cnt_export(cnt1_out)


def _edge_agg(y, e3, n, r, h):
    mesh = plsc.VectorSubcoreMesh(core_axis_name="c", subcore_axis_name="s")
    agg = pl.kernel(
        _agg_body,
        mesh=mesh,
        out_type=jax.ShapeDtypeStruct((_NRANGE, _ACC_REAL, 128),
                                      jnp.float32),
        scratch_types=[
            pltpu.VMEM((3 * _CHUNK,), jnp.int32),
            pltpu.VMEM((_CHUNK // 128, 128), jnp.int32),
            pltpu.VMEM((_CHUNK // 128, 128), jnp.int32),
            pltpu.VMEM((_CHUNK, 128), jnp.float32),
            pltpu.VMEM_SHARED((_ACC_ROWS, 128), jnp.float32),
            pltpu.SemaphoreType.DMA,
            pltpu.SemaphoreType.DMA,
        ],
    )
    acc = agg(y.reshape(n * r, 128), e3)
    return acc.reshape(_NRANGE * _NRG, 3 * 128)[:n]


def _edge_cnt(e3, n, r):
    mesh = plsc.VectorSubcoreMesh(core_axis_name="c", subcore_axis_name="s")
    cntk = pl.kernel(
        _cnt_body,
        mesh=mesh,
        out_type=[
            jax.ShapeDtypeStruct((_CNT_LEN,), jnp.float32),
            jax.ShapeDtypeStruct((_CNT_LEN,), jnp.float32),
        ],
        scratch_types=[
            pltpu.VMEM((3 * _CHUNK,), jnp.int32),
            pltpu.VMEM((_CHUNK // 128, 128), jnp.int32),
            pltpu.VMEM((128,), jnp.float32),
            pltpu.VMEM((1616,), jnp.float32),
            pltpu.VMEM_SHARED((_CNT_LEN,), jnp.float32),
        ],
    )
    cnt0, cnt1 = cntk(e3)
    cnt = jnp.concatenate([cnt0[:_CNT_REAL], cnt1[:_CNT_REAL]])
    return cnt.reshape(_NC * _NHALF, r)[:n]


def _pack_edges(src, dst, etype):
    pad = _EPAD - _E
    z = jnp.zeros((pad,), jnp.int32)
    src = jnp.concatenate([src, z])
    dst = jnp.concatenate([dst, jnp.full((pad,), _N, jnp.int32)])
    etype = jnp.concatenate([etype, z])
    e3 = jnp.stack([src, dst, etype]).reshape(3, _NCH, _CHUNK)
    return e3.transpose(1, 0, 2).reshape(_NCH * 3 * _CHUNK)


def kernel(users_feature, W1, root1, b1, W2, root2, b2, fc1_w, fc1_b,
           fc2_w, fc2_b, edge_index, edge_type, bill_id, user1_id, user2_id):
    n, d = users_feature.shape
    r, _, h = W1.shape
    src = edge_index[0].astype(jnp.int32)
    dst = edge_index[1].astype(jnp.int32)
    etype = edge_type.astype(jnp.int32)
    e3 = _pack_edges(src, dst, etype)

    sel = (jnp.arange(2)[None, :] == (jnp.arange(r) % 2)[:, None])
    sel = sel.astype(jnp.float32)
    w1_cat = jnp.einsum('rdh,rp->drph', W1, sel).reshape(d, r * 2 * h)
    w2_cat = jnp.einsum('rdh,rp->drph', W2, sel).reshape(h, r * 2 * h)

    y1, xr1 = _proj(users_feature, w1_cat, root1, rows_blk=1000)
    cnt = _edge_cnt(e3, n, r)
    acc1 = _edge_agg(y1, e3, n, r, h)
    hfeat = _combine(xr1, acc1, cnt, b1, do_relu=True, rows_blk=1000)

    y2, xr2 = _proj(hfeat, w2_cat, root2, rows_blk=1000)
    acc2 = _edge_agg(y2, e3, n, r, h)
    nodes128 = _combine(xr2, acc2, cnt, b2, do_relu=False, rows_blk=1000,
                        out_pad=128 - h)

    ids = jnp.concatenate([bill_id, user1_id, user2_id]).astype(jnp.int32)
    rows = _id_gather(nodes128, ids)
    return _head(rows, fc1_w, fc1_b, fc2_w, fc2_b)[0, 0]
